# Initial kernel scaffold; baseline (speedup 1.0000x reference)
import jax
import jax.numpy as jnp
from jax.experimental import pallas as pl


def kernel(x, p):
    raise NotImplementedError("write your pallas kernel here")


# speedup vs baseline: 1.0608x; 1.0608x over previous
import jax
import jax.numpy as jnp
from jax.experimental import pallas as pl
from jax.experimental.pallas import tpu as pltpu

_IN = 2048
_OUT = 32768
_K = 655
_ROWS = _OUT // 128
_CBLK = 1024


def _mm_body(x_ref, p_ref, o_ref):
    c = (p_ref[...] > 0.5).astype(jnp.bfloat16)
    xb = x_ref[...].astype(jnp.bfloat16)
    o = jax.lax.dot_general(xb, c, (((1,), (0,)), ((), ())),
                            preferred_element_type=jnp.float32)
    o_ref[...] = o.reshape(_CBLK // 128, 128)


def _topk_body(ov_ref, act_ref, idx_ref, keys_ref):
    v = ov_ref[...].astype(jnp.int32)
    r = jax.lax.broadcasted_iota(jnp.int32, (_ROWS, 128), 0)
    cc = jax.lax.broadcasted_iota(jnp.int32, (_ROWS, 128), 1)
    j = r * 128 + cc
    keys_ref[...] = v * _OUT + (_OUT - 1 - j)

    def step(i, _):
        k = keys_ref[...]
        m = jnp.max(k)
        gidx = (_OUT - 1) - (m & (_OUT - 1))
        idx_ref[pl.ds(i, 1), :] = jnp.full((1, 128), gidx, jnp.int32)
        keys_ref[...] = jnp.where(k == m, -1, k)
        return 0

    jax.lax.fori_loop(0, _K, step, 0)
    act_ref[...] = jnp.where(keys_ref[...] < 0, 1.0, 0.0).astype(jnp.float32)


def kernel(x, p):
    overlap = pl.pallas_call(
        _mm_body,
        grid=(_OUT // _CBLK,),
        in_specs=[
            pl.BlockSpec((1, _IN), lambda i: (0, 0)),
            pl.BlockSpec((_IN, _CBLK), lambda i: (0, i)),
        ],
        out_specs=pl.BlockSpec((_CBLK // 128, 128), lambda i: (i, 0)),
        out_shape=jax.ShapeDtypeStruct((_ROWS, 128), jnp.float32),
    )(x, p)

    act2d, idx2d = pl.pallas_call(
        _topk_body,
        in_specs=[pl.BlockSpec((_ROWS, 128), lambda: (0, 0))],
        out_specs=[
            pl.BlockSpec((_ROWS, 128), lambda: (0, 0)),
            pl.BlockSpec((_K, 128), lambda: (0, 0)),
        ],
        out_shape=[
            jax.ShapeDtypeStruct((_ROWS, 128), jnp.float32),
            jax.ShapeDtypeStruct((_K, 128), jnp.int32),
        ],
        scratch_shapes=[pltpu.VMEM((_ROWS, 128), jnp.int32)],
    )(overlap)

    activation = act2d.reshape(1, _OUT)
    act_indices = idx2d[:, 0].astype(jnp.int64)
    return activation, act_indices


# speedup vs baseline: 1.0647x; 1.0036x over previous
import jax
import jax.numpy as jnp
from jax.experimental import pallas as pl
from jax.experimental.pallas import tpu as pltpu

_IN = 2048
_OUT = 32768
_K = 655
_ROWS = _OUT // 128
_CBLK = 2048


def _mm_body(x_ref, p_ref, o_ref):
    c = (p_ref[...] > 0.5).astype(jnp.bfloat16)
    xb = x_ref[...].astype(jnp.bfloat16)
    o = jax.lax.dot_general(xb, c, (((1,), (0,)), ((), ())),
                            preferred_element_type=jnp.float32)
    o_ref[...] = o.reshape(_CBLK // 128, 128)


def _topk_body(ov_ref, act_ref, idx_ref, keys_ref):
    v = ov_ref[...].astype(jnp.int32)
    r = jax.lax.broadcasted_iota(jnp.int32, (_ROWS, 128), 0)
    cc = jax.lax.broadcasted_iota(jnp.int32, (_ROWS, 128), 1)
    j = r * 128 + cc
    keys_ref[...] = v * _OUT + (_OUT - 1 - j)

    def step(i, _):
        k = keys_ref[...]
        m = jnp.max(k)
        gidx = (_OUT - 1) - (m & (_OUT - 1))
        idx_ref[pl.ds(i, 1), :] = jnp.full((1, 128), gidx, jnp.int32)
        keys_ref[...] = jnp.where(k == m, -1, k)
        return 0

    jax.lax.fori_loop(0, _K, step, 0)
    act_ref[...] = jnp.where(keys_ref[...] < 0, 1.0, 0.0).astype(jnp.float32)


def kernel(x, p):
    overlap = pl.pallas_call(
        _mm_body,
        grid=(_OUT // _CBLK,),
        in_specs=[
            pl.BlockSpec((1, _IN), lambda i: (0, 0)),
            pl.BlockSpec((_IN, _CBLK), lambda i: (0, i)),
        ],
        out_specs=pl.BlockSpec((_CBLK // 128, 128), lambda i: (i, 0)),
        out_shape=jax.ShapeDtypeStruct((_ROWS, 128), jnp.float32),
    )(x, p)

    act2d, idx2d = pl.pallas_call(
        _topk_body,
        in_specs=[pl.BlockSpec((_ROWS, 128), lambda: (0, 0))],
        out_specs=[
            pl.BlockSpec((_ROWS, 128), lambda: (0, 0)),
            pl.BlockSpec((_K, 128), lambda: (0, 0)),
        ],
        out_shape=[
            jax.ShapeDtypeStruct((_ROWS, 128), jnp.float32),
            jax.ShapeDtypeStruct((_K, 128), jnp.int32),
        ],
        scratch_shapes=[pltpu.VMEM((_ROWS, 128), jnp.int32)],
    )(overlap)

    activation = act2d.reshape(1, _OUT)
    act_indices = idx2d[:, 0].astype(jnp.int64)
    return activation, act_indices


# speedup vs baseline: 2.1326x; 2.0030x over previous
import jax
import jax.numpy as jnp
from jax import lax
from jax.experimental import pallas as pl
from jax.experimental.pallas import tpu as pltpu
from jax.experimental.pallas import tpu_sc as plsc

_IN = 2048
_OUT = 32768
_K = 655
_ROWS = _OUT // 128
_CBLK = 2048

_NT = 16
_CH = _OUT // _NT
_NG = _CH // 16
_NCO = 144
_CAND = 688
_OPAD = 704
_DUMP = 700


def _mm_body(x_ref, p_ref, o_ref):
    c = (p_ref[...] > 0.5).astype(jnp.bfloat16)
    xb = x_ref[...].astype(jnp.bfloat16)
    o = jax.lax.dot_general(xb, c, (((1,), (0,)), ((), ())),
                            preferred_element_type=jnp.float32)
    o_ref[...] = o.reshape(_CBLK // 128, 128).astype(jnp.int32)


def _sc_body(sc_ref, act_ref, idx_ref,
             vals_v, act_v, sub_v, co_v, gco_v, fsub_v, fi_v, afine_v,
             cand_v, allcand_v, acnt_v, cn_v, idxA_v, idxB_v, idx1_v,
             val1_v, z_v,
             sh_gco, sh_fi, sh_cnt, sh_cand, sh_out):
    w = lax.axis_index("s")
    it16 = lax.iota(jnp.int32, 16)
    ones = jnp.ones((16,), jnp.int32)
    zeros = jnp.zeros((16,), jnp.int32)

    pltpu.sync_copy(sc_ref.at[pl.ds(w * _CH, _CH)], vals_v)

    def _zfill(i, _):
        z_v[pl.ds(i * 16, 16)] = zeros
        return 0
    lax.fori_loop(0, _OPAD // 16, _zfill, 0)

    def _fa(i, _):
        idxA_v[pl.ds(i * 16, 16)] = i * 16 + it16
        return 0
    lax.fori_loop(0, 8, _fa, 0)
    idxB_v[...] = 128 + it16

    @pl.when(w == 0)
    def _():
        pltpu.sync_copy(z_v, sh_out)
        pltpu.sync_copy(z_v.at[pl.ds(0, _NCO)], sh_gco)
    plsc.subcore_barrier()

    def _zs(i, _):
        sub_v[pl.ds(i * 16, 16)] = zeros
        return 0
    lax.fori_loop(0, _NCO, _zs, 0)

    def _hist(g, _):
        v = vals_v[pl.ds(g * 16, 16)]
        plsc.addupdate_scatter(sub_v, [(v >> 4) * 16 + it16], ones)
        return 0
    lax.fori_loop(0, _NG, _hist, 0)

    def _merge(c, _):
        b = (c * 16 + it16) * 16
        acc = zeros
        for l in range(16):
            acc = acc + plsc.load_gather(sub_v, [b + l])
        co_v[pl.ds(c * 16, 16)] = acc
        return 0
    lax.fori_loop(0, _NCO // 16, _merge, 0)

    pltpu.sync_copy(co_v.at[pl.ds(0, 128)], sh_gco.at[idxA_v], add=True)
    pltpu.sync_copy(co_v.at[pl.ds(128, 16)], sh_gco.at[idxB_v], add=True)
    plsc.subcore_barrier()

    pltpu.sync_copy(sh_gco, gco_v)
    carry = jnp.int32(0)
    cstar = jnp.int32(0)
    sat = jnp.int32(0)
    for c in range(_NCO // 16 - 1, -1, -1):
        vec = gco_v[pl.ds(c * 16, 16)]
        incl = plsc.cumsum(vec)
        tot = jnp.max(incl)
        s = carry + tot - incl
        cond = (s < _K) & (s + vec >= _K)
        cstar = cstar + jnp.sum(jnp.where(cond, c * 16 + it16, 0))
        sat = sat + jnp.sum(jnp.where(cond, s, 0))
        carry = carry + tot

    def _zf(i, _):
        fsub_v[pl.ds(i * 16, 16)] = zeros
        return 0
    lax.fori_loop(0, 16, _zf, 0)

    def _fh(g, _):
        v = vals_v[pl.ds(g * 16, 16)]
        m = (v >> 4) == cstar
        plsc.addupdate_scatter(fsub_v, [(v & 15) * 16 + it16], ones, mask=m)
        return 0
    lax.fori_loop(0, _NG, _fh, 0)

    facc = zeros
    for l in range(16):
        facc = facc + plsc.load_gather(fsub_v, [it16 * 16 + l])
    fi_v[...] = facc
    pltpu.sync_copy(fi_v, sh_fi.at[pl.ds(w * 16, 16)])
    plsc.subcore_barrier()

    pltpu.sync_copy(sh_fi, afine_v)
    gf = zeros
    for r in range(16):
        gf = gf + afine_v[pl.ds(r * 16, 16)]
    inclf = plsc.cumsum(gf)
    totf = jnp.max(inclf)
    sf = sat + totf - inclf
    condf = (sf < _K) & (sf + gf >= _K)
    tstar = jnp.sum(jnp.where(condf, it16, 0))
    gs = jnp.sum(jnp.where(condf, sf, 0))
    ts = cstar * 16 + tstar
    es = _K - gs
    pw0 = plsc.load_gather(afine_v, [it16 * 16 + tstar])
    pw = jnp.sum(jnp.where(it16 < w, pw0, 0))

    def _zc(i, _):
        cand_v[pl.ds(i * 16, 16)] = zeros - 1
        return 0
    lax.fori_loop(0, _CAND // 16, _zc, 0)

    def _main(g, carry2):
        cnt, run = carry2
        v = vals_v[pl.ds(g * 16, 16)]
        jg = w * _CH + g * 16 + it16
        key = (v << 15) | ((_OUT - 1) - jg)
        gt = v > ts
        plsc.store_compressed(cand_v.at[pl.ds(cnt, 16)], key, mask=gt)
        eq = v == ts
        ic = plsc.cumsum(jnp.where(eq, 1, 0))
        pos = pw + run + ic - 1
        sel = eq & (pos < es)
        act_v[pl.ds(g * 16, 16)] = jnp.where(gt | sel, 1.0, 0.0)
        neq = jnp.max(ic)

        @pl.when((neq > 0) & (pw + run < es))
        def _():
            idx1_v[...] = jnp.where(sel, gs + pos, _DUMP)
            val1_v[...] = jnp.where(sel, jg, 0)
            pltpu.sync_copy(val1_v, sh_out.at[idx1_v], add=True)

        return (cnt + jnp.sum(jnp.where(gt, 1, 0)), run + neq)

    cnt, _run = lax.fori_loop(0, _NG, _main, (jnp.int32(0), jnp.int32(0)))
    pltpu.sync_copy(act_v, act_ref.at[pl.ds(w * _CH, _CH)])
    cn_v[...] = zeros + cnt
    pltpu.sync_copy(cn_v, sh_cnt.at[pl.ds(w * 16, 16)])
    pltpu.sync_copy(cand_v, sh_cand.at[pl.ds(w * _CAND, _CAND)])
    plsc.subcore_barrier()

    pltpu.sync_copy(sh_cnt, acnt_v)
    pltpu.sync_copy(sh_cand, allcand_v)
    cnts16 = plsc.load_gather(acnt_v, [it16 * 16])
    mj = (cnt + 15) >> 4

    def _rank(j, _):
        kk = cand_v[pl.ds(j * 16, 16)]
        valid = kk >= 0
        cntv = zeros
        for r in range(16):
            cw = jnp.sum(jnp.where(it16 == r, cnts16, 0))

            def _chunk(ch, acc):
                for rot in range(16):
                    gi = r * _CAND + ch * 16 + ((it16 + rot) & 15)
                    cc = plsc.load_gather(allcand_v, [gi])
                    acc = acc + jnp.where(cc > kk, 1, 0)
                return acc

            cntv = lax.fori_loop(0, (cw + 15) >> 4, _chunk, cntv)
        idx1_v[...] = jnp.where(valid, cntv, _DUMP)
        val1_v[...] = jnp.where(valid, (_OUT - 1) - (kk & (_OUT - 1)), 0)
        pltpu.sync_copy(val1_v, sh_out.at[idx1_v], add=True)
        return 0

    lax.fori_loop(0, mj, _rank, 0)
    plsc.subcore_barrier()

    @pl.when(w == 0)
    def _():
        pltpu.sync_copy(sh_out, idx_ref)


def kernel(x, p):
    scores2d = pl.pallas_call(
        _mm_body,
        grid=(_OUT // _CBLK,),
        in_specs=[
            pl.BlockSpec((1, _IN), lambda i: (0, 0)),
            pl.BlockSpec((_IN, _CBLK), lambda i: (0, i)),
        ],
        out_specs=pl.BlockSpec((_CBLK // 128, 128), lambda i: (i, 0)),
        out_shape=jax.ShapeDtypeStruct((_ROWS, 128), jnp.int32),
    )(x, p)
    scores = scores2d.reshape(_OUT)

    mesh = plsc.VectorSubcoreMesh(core_axis_name="c", subcore_axis_name="s",
                                  num_cores=1)
    act, idx = pl.kernel(
        _sc_body,
        out_type=[
            jax.ShapeDtypeStruct((_OUT,), jnp.float32),
            jax.ShapeDtypeStruct((_OPAD,), jnp.int32),
        ],
        mesh=mesh,
        compiler_params=pltpu.CompilerParams(needs_layout_passes=False),
        scratch_types=[
            pltpu.VMEM((_CH,), jnp.int32),
            pltpu.VMEM((_CH,), jnp.float32),
            pltpu.VMEM((_NCO * 16,), jnp.int32),
            pltpu.VMEM((_NCO,), jnp.int32),
            pltpu.VMEM((_NCO,), jnp.int32),
            pltpu.VMEM((256,), jnp.int32),
            pltpu.VMEM((16,), jnp.int32),
            pltpu.VMEM((256,), jnp.int32),
            pltpu.VMEM((_CAND,), jnp.int32),
            pltpu.VMEM((_NT * _CAND,), jnp.int32),
            pltpu.VMEM((256,), jnp.int32),
            pltpu.VMEM((16,), jnp.int32),
            pltpu.VMEM((128,), jnp.int32),
            pltpu.VMEM((16,), jnp.int32),
            pltpu.VMEM((16,), jnp.int32),
            pltpu.VMEM((16,), jnp.int32),
            pltpu.VMEM((_OPAD,), jnp.int32),
            pltpu.VMEM_SHARED((_NCO,), jnp.int32),
            pltpu.VMEM_SHARED((256,), jnp.int32),
            pltpu.VMEM_SHARED((256,), jnp.int32),
            pltpu.VMEM_SHARED((_NT * _CAND,), jnp.int32),
            pltpu.VMEM_SHARED((_OPAD,), jnp.int32),
        ],
    )(scores)

    activation = act.reshape(1, _OUT)
    act_indices = idx[:_K].astype(jnp.int64)
    return activation, act_indices
